# Initial kernel scaffold; baseline (speedup 1.0000x reference)
import functools

import jax
import jax.numpy as jnp
from jax import lax
from jax.experimental import pallas as pl
from jax.experimental.pallas import tpu as pltpu
from jax.experimental.pallas import tpu_sc as plsc

N = 10000
E = 320000
D = 128
H = 64

NC = 2
NS = 16
L = 128

E_PAD = ((E + NS * L - 1) // (NS * L)) * (NS * L)
PAD = E_PAD - E
ROWS_PER_SUB = E_PAD // NS // L
EDGE_ROWS = E_PAD // L

N_ACC = N + 16
ROWS_N = N // NS
ROWS_ACC = N_ACC // NS

BLK = 1000


_mesh = plsc.VectorSubcoreMesh(core_axis_name="c", subcore_axis_name="s")


@functools.partial(
    pl.kernel,
    out_type=jax.ShapeDtypeStruct((NC, N, H), jnp.float32),
    mesh=_mesh,
    scratch_types=[
        pltpu.VMEM_SHARED((N, H), jnp.float32),
        pltpu.VMEM_SHARED((N_ACC, H), jnp.float32),
        pltpu.VMEM((ROWS_PER_SUB, L), jnp.int32),
        pltpu.VMEM((ROWS_PER_SUB, L), jnp.int32),
        pltpu.VMEM((L, H), jnp.float32),
        pltpu.SemaphoreType.DMA,
    ],
)
def _sc_aggregate(hs_hbm, src_hbm, dst_hbm, zeros_hbm, out_hbm,
                  table, accum, sidx, didx, rows, sem):
    c = lax.axis_index("c")
    s = lax.axis_index("s")
    pltpu.sync_copy(hs_hbm.at[c, pl.ds(s * ROWS_N, ROWS_N)],
                    table.at[pl.ds(s * ROWS_N, ROWS_N)])
    pltpu.sync_copy(zeros_hbm.at[pl.ds(0, ROWS_ACC)],
                    accum.at[pl.ds(s * ROWS_ACC, ROWS_ACC)])
    pltpu.sync_copy(src_hbm.at[pl.ds(s * ROWS_PER_SUB, ROWS_PER_SUB)], sidx)
    pltpu.sync_copy(dst_hbm.at[pl.ds(s * ROWS_PER_SUB, ROWS_PER_SUB)], didx)
    plsc.subcore_barrier()

    def body(j, carry):
        pltpu.async_copy(table.at[sidx.at[j]], rows, sem).wait()
        pltpu.sync_copy(rows, accum.at[didx.at[j]], add=True)
        return carry

    lax.fori_loop(0, ROWS_PER_SUB, body, 0)
    plsc.subcore_barrier()
    pltpu.sync_copy(accum.at[pl.ds(s * ROWS_N, ROWS_N)],
                    out_hbm.at[c, pl.ds(s * ROWS_N, ROWS_N)])


@functools.partial(
    pl.kernel,
    out_type=jax.ShapeDtypeStruct((N, 16), jnp.float32),
    mesh=_mesh,
    scratch_types=[
        pltpu.VMEM_SHARED((N_ACC, 16), jnp.float32),
        pltpu.VMEM((ROWS_PER_SUB, L), jnp.int32),
        pltpu.VMEM((L, 16), jnp.float32),
        pltpu.SemaphoreType.DMA,
    ],
)
def _sc_count(dst_hbm, ones_hbm, zeros_hbm, out_hbm, accum, didx, ones, sem):
    c = lax.axis_index("c")
    s = lax.axis_index("s")

    @pl.when(c == 0)
    def _():
        pltpu.sync_copy(zeros_hbm.at[pl.ds(0, ROWS_ACC)],
                        accum.at[pl.ds(s * ROWS_ACC, ROWS_ACC)])
        pltpu.sync_copy(dst_hbm.at[pl.ds(s * ROWS_PER_SUB, ROWS_PER_SUB)], didx)
        pltpu.sync_copy(ones_hbm, ones)
        plsc.subcore_barrier()

        def body(j, carry):
            pltpu.sync_copy(ones, accum.at[didx.at[j]], add=True)
            return carry

        lax.fori_loop(0, ROWS_PER_SUB, body, 0)
        plsc.subcore_barrier()
        pltpu.sync_copy(accum.at[pl.ds(s * ROWS_N, ROWS_N)],
                        out_hbm.at[pl.ds(s * ROWS_N, ROWS_N)])


def _dense_body(split_out, sums_ref, hs_ref, cnt_ref, wl_ref, wr_ref,
                b_ref, a_ref, out_ref):
    rec = 1.0 / jnp.maximum(cnt_ref[:, 0:1], 1.0)
    m0 = sums_ref[0] * rec
    m1 = sums_ref[1] * rec
    z = (jnp.dot(m0, wl_ref[0:H, :], preferred_element_type=jnp.float32)
         + jnp.dot(m1, wl_ref[H:D, :], preferred_element_type=jnp.float32)
         + jnp.dot(hs_ref[0], wr_ref[0:H, :], preferred_element_type=jnp.float32)
         + jnp.dot(hs_ref[1], wr_ref[H:D, :], preferred_element_type=jnp.float32)
         + b_ref[0:1, :])
    z = jnp.where(z >= 0, z, a_ref[0:1, :] * z)
    if split_out:
        out_ref[0] = z[:, 0:H]
        out_ref[1] = z[:, H:D]
    else:
        out_ref[...] = z


def _make_dense(split_out):
    out_shape = (jax.ShapeDtypeStruct((NC, N, H), jnp.float32) if split_out
                 else jax.ShapeDtypeStruct((N, D), jnp.float32))
    out_spec = (pl.BlockSpec((NC, BLK, H), lambda i: (0, i, 0)) if split_out
                else pl.BlockSpec((BLK, D), lambda i: (i, 0)))
    return pl.pallas_call(
        functools.partial(_dense_body, split_out),
        grid=(N // BLK,),
        in_specs=[
            pl.BlockSpec((NC, BLK, H), lambda i: (0, i, 0)),
            pl.BlockSpec((NC, BLK, H), lambda i: (0, i, 0)),
            pl.BlockSpec((BLK, 16), lambda i: (i, 0)),
            pl.BlockSpec((D, D), lambda i: (0, 0)),
            pl.BlockSpec((D, D), lambda i: (0, 0)),
            pl.BlockSpec((1, D), lambda i: (0, 0)),
            pl.BlockSpec((1, D), lambda i: (0, 0)),
        ],
        out_specs=out_spec,
        out_shape=out_shape,
    )


_dense_split = _make_dense(True)
_dense_full = _make_dense(False)


def kernel(x, edge_index, Wl0, Wr0, b0, a0, Wl1, Wr1, b1, a1,
           Wl2, Wr2, b2, a2, Wl3, Wr3, b3, a3):
    ei = edge_index.astype(jnp.int32)
    src = jnp.concatenate([ei[0], jnp.zeros((PAD,), jnp.int32)]).reshape(EDGE_ROWS, L)
    dst = jnp.concatenate([ei[1], jnp.full((PAD,), N, jnp.int32)]).reshape(EDGE_ROWS, L)
    hs = x.reshape(N, NC, H).transpose(1, 0, 2)
    zeros_h = jnp.zeros((ROWS_ACC + 8, H), jnp.float32)
    zeros_16 = jnp.zeros((ROWS_ACC + 8, 16), jnp.float32)
    ones_16 = jnp.ones((L, 16), jnp.float32)

    cnt = _sc_count(dst, ones_16, zeros_16)

    params = [(Wl0, Wr0, b0, a0), (Wl1, Wr1, b1, a1),
              (Wl2, Wr2, b2, a2), (Wl3, Wr3, b3, a3)]
    for i, (Wl, Wr, b, a) in enumerate(params):
        sums = _sc_aggregate(hs, src, dst, zeros_h)
        dense = _dense_split if i < 3 else _dense_full
        out = dense(sums, hs, cnt, Wl, Wr,
                    b.reshape(1, D), a.reshape(1, D))
        hs = out
    return out


# speedup vs baseline: 3.4877x; 3.4877x over previous
import functools

import jax
import jax.numpy as jnp
from jax import lax
from jax.experimental import pallas as pl
from jax.experimental.pallas import tpu as pltpu
from jax.experimental.pallas import tpu_sc as plsc

N = 10000
E = 320000
D = 128

NC = 2
NS = 16
L = 128

N_PAD = 10112
ROWS_N = N_PAD // NS

E_PAD = 323584
PAD = E_PAD - E
EDGE_ROWS = E_PAD // L
RPW = EDGE_ROWS // (NC * NS)
RPS = EDGE_ROWS // NS

BLK_MID = 1264
BLK_FIN = 1000


_mesh = plsc.VectorSubcoreMesh(core_axis_name="c", subcore_axis_name="s",
                               num_cores=NC, num_subcores=NS)


@functools.partial(
    pl.kernel,
    out_type=jax.ShapeDtypeStruct((NC, N_PAD, D), jnp.float32),
    mesh=_mesh,
    scratch_types=[
        pltpu.VMEM_SHARED((N_PAD, D), jnp.float32),
        pltpu.VMEM((L,), jnp.int32),
        pltpu.VMEM((L,), jnp.int32),
        pltpu.VMEM((L, D), jnp.float32),
        pltpu.SemaphoreType.DMA,
    ],
)
def _sc_aggregate(h_hbm, src_hbm, dst_hbm, zeros_hbm, out_hbm,
                  accum, sidx, didx, rows, sem):
    c = lax.axis_index("c")
    s = lax.axis_index("s")
    pltpu.sync_copy(zeros_hbm, accum.at[pl.ds(s * ROWS_N, ROWS_N)])
    plsc.subcore_barrier()
    base = (c * NS + s) * RPW

    def body(t, carry):
        pltpu.sync_copy(src_hbm.at[base + t], sidx)
        pltpu.sync_copy(dst_hbm.at[base + t], didx)
        pltpu.async_copy(h_hbm.at[sidx], rows, sem).wait()
        pltpu.sync_copy(rows, accum.at[didx], add=True)
        return carry

    lax.fori_loop(0, RPW, body, 0)
    plsc.subcore_barrier()
    pltpu.sync_copy(accum.at[pl.ds(s * ROWS_N, ROWS_N)],
                    out_hbm.at[c, pl.ds(s * ROWS_N, ROWS_N)])


@functools.partial(
    pl.kernel,
    out_type=jax.ShapeDtypeStruct((N_PAD, D), jnp.float32),
    mesh=_mesh,
    scratch_types=[
        pltpu.VMEM_SHARED((N_PAD, D), jnp.float32),
        pltpu.VMEM((L,), jnp.int32),
        pltpu.VMEM((L, D), jnp.float32),
        pltpu.SemaphoreType.DMA,
    ],
)
def _sc_count(dst_hbm, ones_hbm, zeros_hbm, out_hbm, accum, didx, ones, sem):
    c = lax.axis_index("c")
    s = lax.axis_index("s")

    @pl.when(c == 0)
    def _():
        pltpu.sync_copy(zeros_hbm, accum.at[pl.ds(s * ROWS_N, ROWS_N)])
        pltpu.sync_copy(ones_hbm, ones)
        plsc.subcore_barrier()
        base = s * RPS

        def body(t, carry):
            pltpu.sync_copy(dst_hbm.at[base + t], didx)
            pltpu.sync_copy(ones, accum.at[didx], add=True)
            return carry

        lax.fori_loop(0, RPS, body, 0)
        plsc.subcore_barrier()
        pltpu.sync_copy(accum.at[pl.ds(s * ROWS_N, ROWS_N)],
                        out_hbm.at[pl.ds(s * ROWS_N, ROWS_N)])


def _dense_body(sums_ref, h_ref, cnt_ref, wl_ref, wr_ref, b_ref, a_ref,
                out_ref):
    rec = 1.0 / jnp.maximum(cnt_ref[:, 0:1], 1.0)
    m = (sums_ref[0] + sums_ref[1]) * rec
    z = (jnp.dot(m, wl_ref[...], preferred_element_type=jnp.float32)
         + jnp.dot(h_ref[...], wr_ref[...], preferred_element_type=jnp.float32)
         + b_ref[0:1, :])
    out_ref[...] = jnp.where(z >= 0, z, a_ref[0:1, :] * z)


def _make_dense(final):
    blk = BLK_FIN if final else BLK_MID
    grid = (N // blk,) if final else (N_PAD // blk,)
    nrows = N if final else N_PAD
    return pl.pallas_call(
        _dense_body,
        grid=grid,
        in_specs=[
            pl.BlockSpec((NC, blk, D), lambda i: (0, i, 0)),
            pl.BlockSpec((blk, D), lambda i: (i, 0)),
            pl.BlockSpec((blk, 8), lambda i: (i, 0)),
            pl.BlockSpec((D, D), lambda i: (0, 0)),
            pl.BlockSpec((D, D), lambda i: (0, 0)),
            pl.BlockSpec((1, D), lambda i: (0, 0)),
            pl.BlockSpec((1, D), lambda i: (0, 0)),
        ],
        out_specs=pl.BlockSpec((blk, D), lambda i: (i, 0)),
        out_shape=jax.ShapeDtypeStruct((nrows, D), jnp.float32),
    )


_dense_mid = _make_dense(False)
_dense_fin = _make_dense(True)


def kernel(x, edge_index, Wl0, Wr0, b0, a0, Wl1, Wr1, b1, a1,
           Wl2, Wr2, b2, a2, Wl3, Wr3, b3, a3):
    ei = edge_index.astype(jnp.int32)
    src = jnp.concatenate([ei[0], jnp.zeros((PAD,), jnp.int32)]).reshape(EDGE_ROWS, L)
    dst = jnp.concatenate([ei[1], jnp.full((PAD,), N, jnp.int32)]).reshape(EDGE_ROWS, L)
    h = jnp.pad(x, ((0, N_PAD - N), (0, 0)))
    zeros_d = jnp.zeros((ROWS_N, D), jnp.float32)
    ones_d = jnp.ones((L, D), jnp.float32)

    cnt = _sc_count(dst, ones_d, zeros_d)[:, 0:8]
    h, cnt = lax.optimization_barrier((h, cnt))

    params = [(Wl0, Wr0, b0, a0), (Wl1, Wr1, b1, a1),
              (Wl2, Wr2, b2, a2), (Wl3, Wr3, b3, a3)]
    for i, (Wl, Wr, b, a) in enumerate(params):
        sums = _sc_aggregate(h, src, dst, zeros_d)
        dense = _dense_fin if i == 3 else _dense_mid
        h = dense(sums, h, cnt, Wl, Wr, b.reshape(1, D), a.reshape(1, D))
    return h
